# Initial kernel scaffold; baseline (speedup 1.0000x reference)
import jax
import jax.numpy as jnp
from jax.experimental import pallas as pl


def kernel(x, edge_index, batch, device, W1, a_src1, a_dst1, b1, W2, a_src2, a_dst2, b2, W_red, b_red, W_cls, b_cls):
    raise NotImplementedError("write your pallas kernel here")


# speedup vs baseline: 19.2306x; 19.2306x over previous
import jax
import jax.numpy as jnp
from jax import lax
from jax.experimental import pallas as pl
from jax.experimental.pallas import tpu as pltpu
from jax.experimental.pallas import tpu_sc as plsc

N = 10000
E = 320000
F = 128
ROWS_E = E // 128
ROWS_PW = 80
CH = 4
CHB = 2
NEG_SLOPE = 0.2
EPS = 1e-16


def _head1_body(x_ref, w_ref, asrc_ref, adst_ref, h_ref, es_ref, ed_ref, m_ref):
    h = jnp.dot(x_ref[...], w_ref[...], preferred_element_type=jnp.float32)
    h_ref[...] = h
    es = jnp.sum(h * asrc_ref[...][None, :], axis=1)
    ed = jnp.sum(h * adst_ref[...][None, :], axis=1)
    es_ref[...] = es
    ed_ref[...] = ed
    m_ref[...] = jnp.full((16,), jnp.max(es) + jnp.max(ed), jnp.float32)


def _head2_body(oa_ref, ob_ref, b_ref, w_ref, asrc_ref, adst_ref,
                h_ref, es_ref, ed_ref, m_ref):
    hin = jnp.maximum(oa_ref[...] + ob_ref[...] + b_ref[...][None, :], 0.0)
    h = jnp.dot(hin, w_ref[...], preferred_element_type=jnp.float32)
    h_ref[...] = h
    es = jnp.sum(h * asrc_ref[...][None, :], axis=1)
    ed = jnp.sum(h * adst_ref[...][None, :], axis=1)
    es_ref[...] = es
    ed_ref[...] = ed
    m_ref[...] = jnp.full((16,), jnp.max(es) + jnp.max(ed), jnp.float32)


def _tail_body(oa_ref, ob_ref, b_ref, wred_ref, bred_ref, wclsT_ref, bcls_ref,
               y_ref):
    g = jnp.maximum(oa_ref[...] + ob_ref[...] + b_ref[...][None, :], 0.0)
    z = jnp.dot(g, wred_ref[...], preferred_element_type=jnp.float32)[:, 0]
    z = z + bred_ref[0]
    y = jnp.dot(wclsT_ref[...], z, preferred_element_type=jnp.float32)
    y_ref[...] = (y + bcls_ref[...]).reshape(1, 2)


_head1 = pl.pallas_call(
    _head1_body,
    out_shape=[
        jax.ShapeDtypeStruct((N, F), jnp.float32),
        jax.ShapeDtypeStruct((N,), jnp.float32),
        jax.ShapeDtypeStruct((N,), jnp.float32),
        jax.ShapeDtypeStruct((16,), jnp.float32),
    ],
)

_head2 = pl.pallas_call(
    _head2_body,
    out_shape=[
        jax.ShapeDtypeStruct((N, F), jnp.float32),
        jax.ShapeDtypeStruct((N,), jnp.float32),
        jax.ShapeDtypeStruct((N,), jnp.float32),
        jax.ShapeDtypeStruct((16,), jnp.float32),
    ],
)

_tail = pl.pallas_call(
    _tail_body,
    out_shape=jax.ShapeDtypeStruct((1, 2), jnp.float32),
)


_mesh = plsc.VectorSubcoreMesh(core_axis_name="c", subcore_axis_name="s")


def _wid_and_trips():
    c = lax.axis_index("c")
    sid = lax.axis_index("s")
    wid = sid * 2 + c
    trips = jnp.where(wid == 31, 5, ROWS_PW // CH)
    return c, sid, wid, trips


def _pass_a_body(src_hbm, dst_hbm, es_hbm, ed_hbm, m_hbm,
                 al_hbm,
                 es_v, ed_v, s_v, m_v, si_v, di_v, pc_v, z_v, s_sh):
    c = lax.axis_index("c")
    sid = lax.axis_index("s")
    on = c == 0

    @pl.when(on)
    def _():
        pltpu.sync_copy(es_hbm, es_v)
        pltpu.sync_copy(ed_hbm, ed_v)
        pltpu.sync_copy(m_hbm, m_v)

    zero16 = jnp.zeros((16,), jnp.float32)

    def _zb(i, carry):
        z_v[pl.ds(i * 16, 16)] = zero16
        return carry

    lax.fori_loop(0, 64, _zb, 0)

    @pl.when(on & (sid < 10))
    def _():
        pltpu.sync_copy(z_v.at[pl.ds(0, 1000)], s_sh.at[pl.ds(sid * 1000, 1000)])

    plsc.subcore_barrier()

    base_row = sid * 160
    trips = jnp.where(sid == 15, 100 // CH, 160 // CH)

    @pl.when(on)
    def _():
        mvec = m_v[...]

        def _chunk(k, carry):
            row0 = base_row + k * CH
            pltpu.sync_copy(src_hbm.at[pl.ds(row0, CH)], si_v)
            pltpu.sync_copy(dst_hbm.at[pl.ds(row0, CH)], di_v)
            for g in range(CH * 8):
                r, col = g // 8, (g % 8) * 16
                sv = si_v[r, pl.ds(col, 16)]
                dv = di_v[r, pl.ds(col, 16)]
                e = plsc.load_gather(es_v, [sv]) + plsc.load_gather(ed_v, [dv])
                e = jnp.maximum(e, NEG_SLOPE * e)
                pc_v[r, pl.ds(col, 16)] = jnp.exp(e - mvec)
            for r in range(CH):
                pltpu.sync_copy(pc_v.at[r], s_sh.at[di_v.at[r]], add=True)
            return carry

        lax.fori_loop(0, trips, _chunk, 0)

    plsc.subcore_barrier()

    @pl.when(on)
    def _():
        pltpu.sync_copy(s_sh, s_v)
        mvec = m_v[...]

        def _chunk2(k, carry):
            row0 = base_row + k * CH
            pltpu.sync_copy(src_hbm.at[pl.ds(row0, CH)], si_v)
            pltpu.sync_copy(dst_hbm.at[pl.ds(row0, CH)], di_v)
            for g in range(CH * 8):
                r, col = g // 8, (g % 8) * 16
                sv = si_v[r, pl.ds(col, 16)]
                dv = di_v[r, pl.ds(col, 16)]
                e = plsc.load_gather(es_v, [sv]) + plsc.load_gather(ed_v, [dv])
                e = jnp.maximum(e, NEG_SLOPE * e)
                p = jnp.exp(e - mvec)
                den = plsc.load_gather(s_v, [dv])
                pc_v[r, pl.ds(col, 16)] = p / (den + EPS)
            pltpu.sync_copy(pc_v, al_hbm.at[pl.ds(row0, CH)])
            return carry

        lax.fori_loop(0, trips, _chunk2, 0)


_pass_a = pl.kernel(
    _pass_a_body,
    out_type=jax.ShapeDtypeStruct((ROWS_E, 128), jnp.float32),
    mesh=_mesh,
    compiler_params=pltpu.CompilerParams(needs_layout_passes=False),
    scratch_types=[
        pltpu.VMEM((N,), jnp.float32),
        pltpu.VMEM((N,), jnp.float32),
        pltpu.VMEM((N,), jnp.float32),
        pltpu.VMEM((16,), jnp.float32),
        pltpu.VMEM((CH, 128), jnp.int32),
        pltpu.VMEM((CH, 128), jnp.int32),
        pltpu.VMEM((CH, 128), jnp.float32),
        pltpu.VMEM((1024,), jnp.float32),
        pltpu.VMEM_SHARED((N,), jnp.float32),
    ],
)


def _pass_b_body(src_hbm, dst_hbm, al_hbm, h_hbm,
                 oa_hbm, ob_hbm,
                 si_v, di_v, al_v, rows_v, o_sh):
    c, sid, wid, _ = _wid_and_trips()
    trips = jnp.where(wid == 31, 20 // CHB, ROWS_PW // CHB)

    zero16 = jnp.zeros((16,), jnp.float32)
    zi16 = jnp.zeros((16,), jnp.int32)

    def _zb(i, carry):
        rows_v[i // 8, pl.ds((i % 8) * 16, 16)] = zero16
        return carry

    lax.fori_loop(0, CHB * 128 * 8, _zb, 0)

    @pl.when(sid < 10)
    def _():
        for q in range(3):
            pltpu.sync_copy(rows_v, o_sh.at[pl.ds(sid * 1000 + q * 256, 256)])
        pltpu.sync_copy(rows_v.at[pl.ds(0, 232)],
                        o_sh.at[pl.ds(sid * 1000 + 768, 232)])

    plsc.subcore_barrier()

    base_row = wid * ROWS_PW

    def _chunk(k, carry):
        row0 = base_row + k * CHB
        pltpu.sync_copy(src_hbm.at[pl.ds(row0, CHB)], si_v)
        pltpu.sync_copy(dst_hbm.at[pl.ds(row0, CHB)], di_v)
        pltpu.sync_copy(al_hbm.at[pl.ds(row0, CHB)], al_v)
        for r in range(CHB):
            pltpu.sync_copy(h_hbm.at[si_v.at[r]],
                            rows_v.at[pl.ds(r * 128, 128)])
        def _scale(rr, carry2):
            hi = rr // 128
            lo = rr % 128
            asp = plsc.load_gather(al_v, [zi16 + hi, zi16 + lo])
            for cc in range(8):
                rows_v[rr, pl.ds(cc * 16, 16)] = rows_v[rr, pl.ds(cc * 16, 16)] * asp
            return carry2
        lax.fori_loop(0, CHB * 128, _scale, 0)
        for r in range(CHB):
            pltpu.sync_copy(rows_v.at[pl.ds(r * 128, 128)],
                            o_sh.at[di_v.at[r]], add=True)
        return carry

    lax.fori_loop(0, trips, _chunk, 0)
    plsc.subcore_barrier()

    for q in range(4):
        nrow = 256 if q < 3 else 232

        @pl.when(sid < 10)
        def _():
            pltpu.sync_copy(o_sh.at[pl.ds(sid * 1000 + q * 256, nrow)],
                            rows_v.at[pl.ds(0, nrow)])

        @pl.when((sid < 10) & (c == 0))
        def _():
            pltpu.sync_copy(rows_v.at[pl.ds(0, nrow)],
                            oa_hbm.at[pl.ds(sid * 1000 + q * 256, nrow)])

        @pl.when((sid < 10) & (c == 1))
        def _():
            pltpu.sync_copy(rows_v.at[pl.ds(0, nrow)],
                            ob_hbm.at[pl.ds(sid * 1000 + q * 256, nrow)])


_pass_b = pl.kernel(
    _pass_b_body,
    out_type=[
        jax.ShapeDtypeStruct((N, F), jnp.float32),
        jax.ShapeDtypeStruct((N, F), jnp.float32),
    ],
    mesh=_mesh,
    compiler_params=pltpu.CompilerParams(needs_layout_passes=False),
    scratch_types=[
        pltpu.VMEM((CHB, 128), jnp.int32),
        pltpu.VMEM((CHB, 128), jnp.int32),
        pltpu.VMEM((CHB, 128), jnp.float32),
        pltpu.VMEM((CHB * 128, F), jnp.float32),
        pltpu.VMEM_SHARED((N, F), jnp.float32),
    ],
)


def kernel(x, edge_index, batch, device, W1, a_src1, a_dst1, b1,
           W2, a_src2, a_dst2, b2, W_red, b_red, W_cls, b_cls):
    src = edge_index[0].reshape(ROWS_E, 128)
    dst = edge_index[1].reshape(ROWS_E, 128)

    h1, es1, ed1, m1 = _head1(x, W1, a_src1, a_dst1)
    a1 = _pass_a(src, dst, es1, ed1, m1)
    oa1, ob1 = _pass_b(src, dst, a1, h1)

    h2, es2, ed2, m2 = _head2(oa1, ob1, b1, W2, a_src2, a_dst2)
    a2 = _pass_a(src, dst, es2, ed2, m2)
    oa2, ob2 = _pass_b(src, dst, a2, h2)

    return _tail(oa2, ob2, b2, W_red, b_red, W_cls.T, b_cls)


# speedup vs baseline: 27.3158x; 1.4204x over previous
import jax
import jax.numpy as jnp
from jax import lax
from jax.experimental import pallas as pl
from jax.experimental.pallas import tpu as pltpu
from jax.experimental.pallas import tpu_sc as plsc

N = 10000
E = 320000
F = 128
ROWS_E = E // 128
ROWS_PW = 80
CH = 20
NEG_SLOPE = 0.2
EPS = 1e-16


def _head1_body(x_ref, w_ref, asrc_ref, adst_ref, h_ref, es_ref, ed_ref, m_ref):
    h = jnp.dot(x_ref[...], w_ref[...], preferred_element_type=jnp.float32)
    h_ref[...] = h
    es = jnp.sum(h * asrc_ref[...][None, :], axis=1)
    ed = jnp.sum(h * adst_ref[...][None, :], axis=1)
    es_ref[...] = es
    ed_ref[...] = ed
    m_ref[...] = jnp.full((16,), jnp.max(es) + jnp.max(ed), jnp.float32)


def _head2_body(oa_ref, ob_ref, b_ref, w_ref, asrc_ref, adst_ref,
                h_ref, es_ref, ed_ref, m_ref):
    hin = jnp.maximum(oa_ref[...] + ob_ref[...] + b_ref[...][None, :], 0.0)
    h = jnp.dot(hin, w_ref[...], preferred_element_type=jnp.float32)
    h_ref[...] = h
    es = jnp.sum(h * asrc_ref[...][None, :], axis=1)
    ed = jnp.sum(h * adst_ref[...][None, :], axis=1)
    es_ref[...] = es
    ed_ref[...] = ed
    m_ref[...] = jnp.full((16,), jnp.max(es) + jnp.max(ed), jnp.float32)


def _tail_body(oa_ref, ob_ref, b_ref, wred_ref, bred_ref, wclsT_ref, bcls_ref,
               y_ref):
    g = jnp.maximum(oa_ref[...] + ob_ref[...] + b_ref[...][None, :], 0.0)
    z = jnp.dot(g, wred_ref[...], preferred_element_type=jnp.float32)[:, 0]
    z = z + bred_ref[0]
    y = jnp.dot(wclsT_ref[...], z, preferred_element_type=jnp.float32)
    y_ref[...] = (y + bcls_ref[...]).reshape(1, 2)


_head1 = pl.pallas_call(
    _head1_body,
    out_shape=[
        jax.ShapeDtypeStruct((N, F), jnp.float32),
        jax.ShapeDtypeStruct((N,), jnp.float32),
        jax.ShapeDtypeStruct((N,), jnp.float32),
        jax.ShapeDtypeStruct((16,), jnp.float32),
    ],
)

_head2 = pl.pallas_call(
    _head2_body,
    out_shape=[
        jax.ShapeDtypeStruct((N, F), jnp.float32),
        jax.ShapeDtypeStruct((N,), jnp.float32),
        jax.ShapeDtypeStruct((N,), jnp.float32),
        jax.ShapeDtypeStruct((16,), jnp.float32),
    ],
)

_tail = pl.pallas_call(
    _tail_body,
    out_shape=jax.ShapeDtypeStruct((1, 2), jnp.float32),
)


_mesh = plsc.VectorSubcoreMesh(core_axis_name="c", subcore_axis_name="s")


def _wid_and_trips():
    c = lax.axis_index("c")
    sid = lax.axis_index("s")
    wid = sid * 2 + c
    trips = jnp.where(wid == 31, 5, ROWS_PW // CH)
    return c, sid, wid, trips


def _pass_a_body(sd_hbm, es_hbm, ed_hbm, m_hbm,
                 al_hbm,
                 es_v, ed_v, s_v, m_v, sd_v, pc_v, z_v, ssem, s_sh):
    c = lax.axis_index("c")
    sid = lax.axis_index("s")
    on = c == 0

    @pl.when(on)
    def _():
        pltpu.sync_copy(es_hbm, es_v)
        pltpu.sync_copy(ed_hbm, ed_v)
        pltpu.sync_copy(m_hbm, m_v)

    zero16 = jnp.zeros((16,), jnp.float32)

    def _zb(i, carry):
        z_v[pl.ds(i * 16, 16)] = zero16
        return carry

    lax.fori_loop(0, 64, _zb, 0)

    @pl.when(on & (sid < 10))
    def _():
        pltpu.sync_copy(z_v.at[pl.ds(0, 1000)], s_sh.at[pl.ds(sid * 1000, 1000)])

    plsc.subcore_barrier()

    base_row = sid * 160
    trips = jnp.where(sid == 15, 100 // CH, 160 // CH)

    @pl.when(on)
    def _():
        mvec = m_v[...]

        def _chunk(k, carry):
            row0 = base_row + k * CH
            pltpu.sync_copy(sd_hbm.at[pl.ds(row0, CH)], sd_v)
            for g in range(CH * 8):
                r, col = g // 8, (g % 8) * 16
                sv = sd_v[r, 0, pl.ds(col, 16)]
                dv = sd_v[r, 1, pl.ds(col, 16)]
                e = plsc.load_gather(es_v, [sv]) + plsc.load_gather(ed_v, [dv])
                e = jnp.maximum(e, NEG_SLOPE * e)
                pc_v[r, 0, pl.ds(col, 16)] = jnp.exp(e - mvec)
            for r in range(CH):
                pltpu.async_copy(pc_v.at[r, 0], s_sh.at[sd_v.at[r, 1]], ssem,
                                 add=True)
            for r in range(CH):
                pltpu.make_async_copy(pc_v.at[r, 0], s_sh.at[sd_v.at[r, 1]],
                                      ssem).wait()
            return carry

        lax.fori_loop(0, trips, _chunk, 0)

    plsc.subcore_barrier()

    @pl.when(on)
    def _():
        pltpu.sync_copy(s_sh, s_v)
        mvec = m_v[...]

        def _chunk2(k, carry):
            row0 = base_row + k * CH
            pltpu.sync_copy(sd_hbm.at[pl.ds(row0, CH)], sd_v)
            for g in range(CH * 8):
                r, col = g // 8, (g % 8) * 16
                sv = sd_v[r, 0, pl.ds(col, 16)]
                dv = sd_v[r, 1, pl.ds(col, 16)]
                e = plsc.load_gather(es_v, [sv]) + plsc.load_gather(ed_v, [dv])
                e = jnp.maximum(e, NEG_SLOPE * e)
                p = jnp.exp(e - mvec)
                den = plsc.load_gather(s_v, [dv])
                pc_v[r, 0, pl.ds(col, 16)] = p / (den + EPS)
            pltpu.sync_copy(pc_v, al_hbm.at[pl.ds(row0, CH)])
            return carry

        lax.fori_loop(0, trips, _chunk2, 0)


_pass_a = pl.kernel(
    _pass_a_body,
    out_type=jax.ShapeDtypeStruct((ROWS_E, 1, 128), jnp.float32),
    mesh=_mesh,
    compiler_params=pltpu.CompilerParams(needs_layout_passes=False),
    scratch_types=[
        pltpu.VMEM((N,), jnp.float32),
        pltpu.VMEM((N,), jnp.float32),
        pltpu.VMEM((N,), jnp.float32),
        pltpu.VMEM((16,), jnp.float32),
        pltpu.VMEM((CH, 2, 128), jnp.int32),
        pltpu.VMEM((CH, 1, 128), jnp.float32),
        pltpu.VMEM((1024,), jnp.float32),
        pltpu.SemaphoreType.DMA,
        pltpu.VMEM_SHARED((N,), jnp.float32),
    ],
)


def _pass_b_body(sd_hbm, al_hbm, h_hbm,
                 oa_hbm, ob_hbm,
                 sd0_v, sd1_v, al0_v, al1_v, rows0_v, rows1_v,
                 gsem0, gsem1, o_sh):
    c, sid, wid, _ = _wid_and_trips()
    trips2 = jnp.where(wid == 31, 10, ROWS_PW // 2)

    zero16 = jnp.zeros((16,), jnp.float32)
    zi16 = jnp.zeros((16,), jnp.int32)

    def _zb(i, carry):
        rows0_v[i // 8, pl.ds((i % 8) * 16, 16)] = zero16
        return carry

    lax.fori_loop(0, 128 * 8, _zb, 0)

    @pl.when(sid < 10)
    def _():
        for q in range(7):
            pltpu.sync_copy(rows0_v, o_sh.at[pl.ds(sid * 1000 + q * 128, 128)])
        pltpu.sync_copy(rows0_v.at[pl.ds(0, 104)],
                        o_sh.at[pl.ds(sid * 1000 + 896, 104)])

    plsc.subcore_barrier()

    base_row = wid * ROWS_PW
    bufs = ((sd0_v, al0_v, rows0_v, gsem0),
            (sd1_v, al1_v, rows1_v, gsem1))

    def _fetch(row, b):
        sd_v, al_v, rows_v, gsem = bufs[b]
        pltpu.sync_copy(sd_hbm.at[pl.ds(row, 1)], sd_v)
        pltpu.sync_copy(al_hbm.at[pl.ds(row, 1)], al_v)
        pltpu.async_copy(h_hbm.at[sd_v.at[0, 0]], rows_v, gsem)

    def _process(b):
        sd_v, al_v, rows_v, gsem = bufs[b]
        pltpu.make_async_copy(h_hbm.at[sd_v.at[0, 0]], rows_v, gsem).wait()

        def _scale(rr, carry2):
            asp = plsc.load_gather(al_v, [zi16, zi16, zi16 + rr])
            for cc in range(8):
                rows_v[rr, pl.ds(cc * 16, 16)] = (
                    rows_v[rr, pl.ds(cc * 16, 16)] * asp)
            return carry2

        lax.fori_loop(0, 128, _scale, 0)
        pltpu.sync_copy(rows_v, o_sh.at[sd_v.at[0, 1]], add=True)

    _fetch(base_row, 0)

    def _pair(j, carry):
        _fetch(base_row + 2 * j + 1, 1)
        _process(0)

        @pl.when(j + 1 < trips2)
        def _():
            _fetch(base_row + 2 * j + 2, 0)

        _process(1)
        return carry

    lax.fori_loop(0, trips2, _pair, 0)
    plsc.subcore_barrier()

    for q in range(8):
        nrow = 128 if q < 7 else 104

        @pl.when(sid < 10)
        def _():
            pltpu.sync_copy(o_sh.at[pl.ds(sid * 1000 + q * 128, nrow)],
                            rows0_v.at[pl.ds(0, nrow)])

        @pl.when((sid < 10) & (c == 0))
        def _():
            pltpu.sync_copy(rows0_v.at[pl.ds(0, nrow)],
                            oa_hbm.at[pl.ds(sid * 1000 + q * 128, nrow)])

        @pl.when((sid < 10) & (c == 1))
        def _():
            pltpu.sync_copy(rows0_v.at[pl.ds(0, nrow)],
                            ob_hbm.at[pl.ds(sid * 1000 + q * 128, nrow)])


_pass_b = pl.kernel(
    _pass_b_body,
    out_type=[
        jax.ShapeDtypeStruct((N, F), jnp.float32),
        jax.ShapeDtypeStruct((N, F), jnp.float32),
    ],
    mesh=_mesh,
    compiler_params=pltpu.CompilerParams(needs_layout_passes=False),
    scratch_types=[
        pltpu.VMEM((1, 2, 128), jnp.int32),
        pltpu.VMEM((1, 2, 128), jnp.int32),
        pltpu.VMEM((1, 1, 128), jnp.float32),
        pltpu.VMEM((1, 1, 128), jnp.float32),
        pltpu.VMEM((128, F), jnp.float32),
        pltpu.VMEM((128, F), jnp.float32),
        pltpu.SemaphoreType.DMA,
        pltpu.SemaphoreType.DMA,
        pltpu.VMEM_SHARED((N, F), jnp.float32),
    ],
)


def kernel(x, edge_index, batch, device, W1, a_src1, a_dst1, b1,
           W2, a_src2, a_dst2, b2, W_red, b_red, W_cls, b_cls):
    sd = jnp.transpose(edge_index.reshape(2, ROWS_E, 128), (1, 0, 2))

    h1, es1, ed1, m1 = _head1(x, W1, a_src1, a_dst1)
    a1 = _pass_a(sd, es1, ed1, m1)
    oa1, ob1 = _pass_b(sd, a1, h1)

    h2, es2, ed2, m2 = _head2(oa1, ob1, b1, W2, a_src2, a_dst2)
    a2 = _pass_a(sd, es2, ed2, m2)
    oa2, ob2 = _pass_b(sd, a2, h2)

    return _tail(oa2, ob2, b2, W_red, b_red, W_cls.T, b_cls)


# speedup vs baseline: 33.6562x; 1.2321x over previous
import jax
import jax.numpy as jnp
from jax import lax
from jax.experimental import pallas as pl
from jax.experimental.pallas import tpu as pltpu
from jax.experimental.pallas import tpu_sc as plsc

N = 10000
E = 320000
F = 128
ROWS_E = E // 128
ROWS_PW = 80
CH = 20
NEG_SLOPE = 0.2
EPS = 1e-16


def _head1_body(x_ref, w_ref, asrc_ref, adst_ref, h_ref, es_ref, ed_ref, m_ref):
    h = jnp.dot(x_ref[...], w_ref[...], preferred_element_type=jnp.float32)
    h_ref[...] = h
    es = jnp.sum(h * asrc_ref[...][None, :], axis=1)
    ed = jnp.sum(h * adst_ref[...][None, :], axis=1)
    es_ref[...] = es
    ed_ref[...] = ed
    m_ref[...] = jnp.full((16,), jnp.max(es) + jnp.max(ed), jnp.float32)


def _head2_body(oa_ref, ob_ref, b_ref, w_ref, asrc_ref, adst_ref,
                h_ref, es_ref, ed_ref, m_ref):
    hin = jnp.maximum(oa_ref[...] + ob_ref[...] + b_ref[...][None, :], 0.0)
    h = jnp.dot(hin, w_ref[...], preferred_element_type=jnp.float32)
    h_ref[...] = h
    es = jnp.sum(h * asrc_ref[...][None, :], axis=1)
    ed = jnp.sum(h * adst_ref[...][None, :], axis=1)
    es_ref[...] = es
    ed_ref[...] = ed
    m_ref[...] = jnp.full((16,), jnp.max(es) + jnp.max(ed), jnp.float32)


def _tail_body(oa_ref, ob_ref, b_ref, wred_ref, bred_ref, wclsT_ref, bcls_ref,
               y_ref):
    g = jnp.maximum(oa_ref[...] + ob_ref[...] + b_ref[...][None, :], 0.0)
    z = jnp.dot(g, wred_ref[...], preferred_element_type=jnp.float32)[:, 0]
    z = z + bred_ref[0]
    y = jnp.dot(wclsT_ref[...], z, preferred_element_type=jnp.float32)
    y_ref[...] = (y + bcls_ref[...]).reshape(1, 2)


_head1 = pl.pallas_call(
    _head1_body,
    out_shape=[
        jax.ShapeDtypeStruct((N, F), jnp.float32),
        jax.ShapeDtypeStruct((N,), jnp.float32),
        jax.ShapeDtypeStruct((N,), jnp.float32),
        jax.ShapeDtypeStruct((16,), jnp.float32),
    ],
)

_head2 = pl.pallas_call(
    _head2_body,
    out_shape=[
        jax.ShapeDtypeStruct((N, F), jnp.float32),
        jax.ShapeDtypeStruct((N,), jnp.float32),
        jax.ShapeDtypeStruct((N,), jnp.float32),
        jax.ShapeDtypeStruct((16,), jnp.float32),
    ],
)

_tail = pl.pallas_call(
    _tail_body,
    out_shape=jax.ShapeDtypeStruct((1, 2), jnp.float32),
)


_mesh = plsc.VectorSubcoreMesh(core_axis_name="c", subcore_axis_name="s")


def _wid_and_trips():
    c = lax.axis_index("c")
    sid = lax.axis_index("s")
    wid = sid * 2 + c
    trips = jnp.where(wid == 31, 5, ROWS_PW // CH)
    return c, sid, wid, trips


def _pass_a_body(sd_hbm, es_hbm, ed_hbm, m_hbm,
                 al_hbm,
                 es_v, ed_v, s_v, m_v, sd_v, pc_v, z_v, ssem, s_sh):
    c = lax.axis_index("c")
    sid = lax.axis_index("s")
    on = c == 0

    @pl.when(on)
    def _():
        pltpu.sync_copy(es_hbm, es_v)
        pltpu.sync_copy(ed_hbm, ed_v)
        pltpu.sync_copy(m_hbm, m_v)

    zero16 = jnp.zeros((16,), jnp.float32)

    def _zb(i, carry):
        z_v[pl.ds(i * 16, 16)] = zero16
        return carry

    lax.fori_loop(0, 64, _zb, 0)

    @pl.when(on & (sid < 10))
    def _():
        pltpu.sync_copy(z_v.at[pl.ds(0, 1000)], s_sh.at[pl.ds(sid * 1000, 1000)])

    plsc.subcore_barrier()

    base_row = sid * 160
    trips = jnp.where(sid == 15, 100 // CH, 160 // CH)

    @pl.when(on)
    def _():
        mvec = m_v[...]

        def _chunk(k, carry):
            row0 = base_row + k * CH
            pltpu.sync_copy(sd_hbm.at[pl.ds(row0, CH)], sd_v)
            for g in range(CH * 8):
                r, col = g // 8, (g % 8) * 16
                sv = sd_v[r, 0, pl.ds(col, 16)]
                dv = sd_v[r, 1, pl.ds(col, 16)]
                e = plsc.load_gather(es_v, [sv]) + plsc.load_gather(ed_v, [dv])
                e = jnp.maximum(e, NEG_SLOPE * e)
                pc_v[r, 0, pl.ds(col, 16)] = jnp.exp(e - mvec)
            for r in range(CH):
                pltpu.async_copy(pc_v.at[r, 0], s_sh.at[sd_v.at[r, 1]], ssem,
                                 add=True)
            for r in range(CH):
                pltpu.make_async_copy(pc_v.at[r, 0], s_sh.at[sd_v.at[r, 1]],
                                      ssem).wait()
            return carry

        lax.fori_loop(0, trips, _chunk, 0)

    plsc.subcore_barrier()

    @pl.when(on)
    def _():
        pltpu.sync_copy(s_sh, s_v)
        mvec = m_v[...]

        def _chunk2(k, carry):
            row0 = base_row + k * CH
            pltpu.sync_copy(sd_hbm.at[pl.ds(row0, CH)], sd_v)
            for g in range(CH * 8):
                r, col = g // 8, (g % 8) * 16
                sv = sd_v[r, 0, pl.ds(col, 16)]
                dv = sd_v[r, 1, pl.ds(col, 16)]
                e = plsc.load_gather(es_v, [sv]) + plsc.load_gather(ed_v, [dv])
                e = jnp.maximum(e, NEG_SLOPE * e)
                p = jnp.exp(e - mvec)
                den = plsc.load_gather(s_v, [dv])
                pc_v[r, 0, pl.ds(col, 16)] = p / (den + EPS)
            pltpu.sync_copy(pc_v, al_hbm.at[pl.ds(row0, CH)])
            return carry

        lax.fori_loop(0, trips, _chunk2, 0)


_pass_a = pl.kernel(
    _pass_a_body,
    out_type=jax.ShapeDtypeStruct((ROWS_E, 1, 128), jnp.float32),
    mesh=_mesh,
    compiler_params=pltpu.CompilerParams(needs_layout_passes=False),
    scratch_types=[
        pltpu.VMEM((N,), jnp.float32),
        pltpu.VMEM((N,), jnp.float32),
        pltpu.VMEM((N,), jnp.float32),
        pltpu.VMEM((16,), jnp.float32),
        pltpu.VMEM((CH, 2, 128), jnp.int32),
        pltpu.VMEM((CH, 1, 128), jnp.float32),
        pltpu.VMEM((1024,), jnp.float32),
        pltpu.SemaphoreType.DMA,
        pltpu.VMEM_SHARED((N,), jnp.float32),
    ],
)


def _pass_b_body(sd_hbm, al_hbm, h_hbm,
                 oa_hbm, ob_hbm,
                 sd8_v, al8_v, rows0_v, rows1_v,
                 gsem0, gsem1, ssem0, ssem1, o_sh):
    c = lax.axis_index("c")
    sid = lax.axis_index("s")
    wid = sid * 2 + c
    nblk = jnp.where(wid < 24, 10, 9)
    base_blk = jnp.where(wid < 24, wid * 10, 240 + (wid - 24) * 9)

    zero16 = jnp.zeros((16,), jnp.float32)
    zi16 = jnp.zeros((16,), jnp.int32)

    def _zb(i, carry):
        rows0_v[i // 8, pl.ds((i % 8) * 16, 16)] = zero16
        return carry

    lax.fori_loop(0, 128 * 8, _zb, 0)

    @pl.when(sid < 10)
    def _():
        for q in range(7):
            pltpu.sync_copy(rows0_v, o_sh.at[pl.ds(sid * 1000 + q * 128, 128)])
        pltpu.sync_copy(rows0_v.at[pl.ds(0, 104)],
                        o_sh.at[pl.ds(sid * 1000 + 896, 104)])

    plsc.subcore_barrier()

    rows = (rows0_v, rows1_v)
    gsems = (gsem0, gsem1)
    ssems = (ssem0, ssem1)

    def _fetch_blk(j, par):
        row0 = (base_blk + j) * 8
        pltpu.sync_copy(sd_hbm.at[pl.ds(row0, 8)], sd8_v.at[par])
        pltpu.sync_copy(al_hbm.at[pl.ds(row0, 8)], al8_v.at[par])

    def _gather(par, rr, b):
        pltpu.async_copy(h_hbm.at[sd8_v.at[par, rr, 0]], rows[b], gsems[b])

    def _wait_gather(par, rr, b):
        pltpu.make_async_copy(h_hbm.at[sd8_v.at[par, rr, 0]], rows[b],
                              gsems[b]).wait()

    def _scatter(par, rr, b):
        pltpu.async_copy(rows[b], o_sh.at[sd8_v.at[par, rr, 1]], ssems[b],
                         add=True)

    def _wait_scatter(par, rr, b):
        pltpu.make_async_copy(rows[b], o_sh.at[sd8_v.at[par, rr, 1]],
                              ssems[b]).wait()

    def _scale(par, rr, b):
        rv = rows[b]
        p16 = zi16 + par
        r16 = zi16 + rr

        def _row(q, carry2):
            asp = plsc.load_gather(al8_v, [p16, r16, zi16, zi16 + q])
            for cc in range(8):
                rv[q, pl.ds(cc * 16, 16)] = rv[q, pl.ds(cc * 16, 16)] * asp
            return carry2

        lax.fori_loop(0, 128, _row, 0)

    _fetch_blk(0, 0)
    _gather(0, 0, 0)

    def _blk(j, carry):
        par = j % 2
        nxt = 1 - par

        @pl.when(j + 1 < nblk)
        def _():
            _fetch_blk(j + 1, nxt)

        for rr in range(8):
            b = rr % 2
            bo = 1 - b
            _wait_gather(par, rr, b)
            @pl.when((j > 0) | (rr > 0))
            def _():
                _wait_scatter(par, rr, bo)
            if rr < 7:
                _gather(par, rr + 1, bo)
            else:
                @pl.when(j + 1 < nblk)
                def _():
                    _gather(nxt, 0, bo)
            _scale(par, rr, b)
            _scatter(par, rr, b)
        return carry

    lax.fori_loop(0, nblk, _blk, 0)
    _wait_scatter(0, 0, 1)

    @pl.when(wid == 31)
    def _():
        pltpu.sync_copy(sd_hbm.at[pl.ds(2496, 4)], sd8_v.at[0, pl.ds(0, 4)])
        pltpu.sync_copy(al_hbm.at[pl.ds(2496, 4)], al8_v.at[0, pl.ds(0, 4)])
        for rr in range(4):
            pltpu.sync_copy(h_hbm.at[sd8_v.at[0, rr, 0]], rows0_v)
            _scale(0, rr, 0)
            pltpu.sync_copy(rows0_v, o_sh.at[sd8_v.at[0, rr, 1]], add=True)

    plsc.subcore_barrier()

    for q in range(8):
        nrow = 128 if q < 7 else 104

        @pl.when(sid < 10)
        def _():
            pltpu.sync_copy(o_sh.at[pl.ds(sid * 1000 + q * 128, nrow)],
                            rows0_v.at[pl.ds(0, nrow)])

        @pl.when((sid < 10) & (c == 0))
        def _():
            pltpu.sync_copy(rows0_v.at[pl.ds(0, nrow)],
                            oa_hbm.at[pl.ds(sid * 1000 + q * 128, nrow)])

        @pl.when((sid < 10) & (c == 1))
        def _():
            pltpu.sync_copy(rows0_v.at[pl.ds(0, nrow)],
                            ob_hbm.at[pl.ds(sid * 1000 + q * 128, nrow)])


_pass_b = pl.kernel(
    _pass_b_body,
    out_type=[
        jax.ShapeDtypeStruct((N, F), jnp.float32),
        jax.ShapeDtypeStruct((N, F), jnp.float32),
    ],
    mesh=_mesh,
    compiler_params=pltpu.CompilerParams(needs_layout_passes=False),
    scratch_types=[
        pltpu.VMEM((2, 8, 2, 128), jnp.int32),
        pltpu.VMEM((2, 8, 1, 128), jnp.float32),
        pltpu.VMEM((128, F), jnp.float32),
        pltpu.VMEM((128, F), jnp.float32),
        pltpu.SemaphoreType.DMA,
        pltpu.SemaphoreType.DMA,
        pltpu.SemaphoreType.DMA,
        pltpu.SemaphoreType.DMA,
        pltpu.VMEM_SHARED((N, F), jnp.float32),
    ],
)


def kernel(x, edge_index, batch, device, W1, a_src1, a_dst1, b1,
           W2, a_src2, a_dst2, b2, W_red, b_red, W_cls, b_cls):
    sd = jnp.transpose(edge_index.reshape(2, ROWS_E, 128), (1, 0, 2))

    h1, es1, ed1, m1 = _head1(x, W1, a_src1, a_dst1)
    a1 = _pass_a(sd, es1, ed1, m1)
    oa1, ob1 = _pass_b(sd, a1, h1)

    h2, es2, ed2, m2 = _head2(oa1, ob1, b1, W2, a_src2, a_dst2)
    a2 = _pass_a(sd, es2, ed2, m2)
    oa2, ob2 = _pass_b(sd, a2, h2)

    return _tail(oa2, ob2, b2, W_red, b_red, W_cls.T, b_cls)


# speedup vs baseline: 39.7843x; 1.1821x over previous
import jax
import jax.numpy as jnp
from jax import lax
from jax.experimental import pallas as pl
from jax.experimental.pallas import tpu as pltpu
from jax.experimental.pallas import tpu_sc as plsc

N = 10000
E = 320000
F = 128
ROWS_E = E // 128
ROWS_PW = 80
CH = 20
NEG_SLOPE = 0.2
EPS = 1e-16


def _head1_body(x_ref, w_ref, asrc_ref, adst_ref, h_ref, es_ref, ed_ref, m_ref):
    h = jnp.dot(x_ref[...], w_ref[...], preferred_element_type=jnp.float32)
    h_ref[...] = h
    es = jnp.sum(h * asrc_ref[...][None, :], axis=1)
    ed = jnp.sum(h * adst_ref[...][None, :], axis=1)
    es_ref[...] = es
    ed_ref[...] = ed
    m_ref[...] = jnp.full((16,), jnp.max(es) + jnp.max(ed), jnp.float32)


def _head2_body(oa_ref, ob_ref, b_ref, w_ref, asrc_ref, adst_ref,
                h_ref, es_ref, ed_ref, m_ref):
    hin = jnp.maximum(oa_ref[...] + ob_ref[...] + b_ref[...][None, :], 0.0)
    h = jnp.dot(hin, w_ref[...], preferred_element_type=jnp.float32)
    h_ref[...] = h
    es = jnp.sum(h * asrc_ref[...][None, :], axis=1)
    ed = jnp.sum(h * adst_ref[...][None, :], axis=1)
    es_ref[...] = es
    ed_ref[...] = ed
    m_ref[...] = jnp.full((16,), jnp.max(es) + jnp.max(ed), jnp.float32)


def _tail_body(oa_ref, ob_ref, b_ref, wred_ref, bred_ref, wclsT_ref, bcls_ref,
               y_ref):
    g = jnp.maximum(oa_ref[...] + ob_ref[...] + b_ref[...][None, :], 0.0)
    z = jnp.dot(g, wred_ref[...], preferred_element_type=jnp.float32)[:, 0]
    z = z + bred_ref[0]
    y = jnp.dot(wclsT_ref[...], z, preferred_element_type=jnp.float32)
    y_ref[...] = (y + bcls_ref[...]).reshape(1, 2)


_head1 = pl.pallas_call(
    _head1_body,
    out_shape=[
        jax.ShapeDtypeStruct((N, F), jnp.float32),
        jax.ShapeDtypeStruct((N,), jnp.float32),
        jax.ShapeDtypeStruct((N,), jnp.float32),
        jax.ShapeDtypeStruct((16,), jnp.float32),
    ],
)

_head2 = pl.pallas_call(
    _head2_body,
    out_shape=[
        jax.ShapeDtypeStruct((N, F), jnp.float32),
        jax.ShapeDtypeStruct((N,), jnp.float32),
        jax.ShapeDtypeStruct((N,), jnp.float32),
        jax.ShapeDtypeStruct((16,), jnp.float32),
    ],
)

_tail = pl.pallas_call(
    _tail_body,
    out_shape=jax.ShapeDtypeStruct((1, 2), jnp.float32),
)


_mesh = plsc.VectorSubcoreMesh(core_axis_name="c", subcore_axis_name="s")


def _wid_and_trips():
    c = lax.axis_index("c")
    sid = lax.axis_index("s")
    wid = sid * 2 + c
    trips = jnp.where(wid == 31, 5, ROWS_PW // CH)
    return c, sid, wid, trips


def _pass_a_body(sd_hbm, es_hbm, ed_hbm, m_hbm,
                 al_hbm, sa_hbm, sb_hbm,
                 es_v, ed_v, m_v, sd_all, p_all, z_v, ssem, bsem, s_sh):
    c = lax.axis_index("c")
    sid = lax.axis_index("s")
    wid = sid * 2 + c
    nrows = jnp.where(wid < 24, 80, jnp.where(wid < 31, 72, 76))
    base = jnp.where(wid < 24, wid * 80, 1920 + (wid - 24) * 72)
    nchunk = jnp.where(wid < 24, 10, 9)

    pltpu.sync_copy(es_hbm, es_v)
    pltpu.sync_copy(ed_hbm, ed_v)
    pltpu.sync_copy(m_hbm, m_v)
    pltpu.sync_copy(sd_hbm.at[pl.ds(base, 72)], sd_all.at[pl.ds(0, 72)])

    @pl.when(wid < 24)
    def _():
        pltpu.sync_copy(sd_hbm.at[pl.ds(base + 72, 8)],
                        sd_all.at[pl.ds(72, 8)])

    @pl.when(wid == 31)
    def _():
        pltpu.sync_copy(sd_hbm.at[pl.ds(2496, 4)], sd_all.at[pl.ds(72, 4)])

    zero16 = jnp.zeros((16,), jnp.float32)

    def _zb(i, carry):
        z_v[pl.ds(i * 16, 16)] = zero16
        return carry

    lax.fori_loop(0, 64, _zb, 0)

    @pl.when(sid < 10)
    def _():
        pltpu.sync_copy(z_v.at[pl.ds(0, 1000)], s_sh.at[pl.ds(sid * 1000, 1000)])

    plsc.subcore_barrier()

    mvec = m_v[...]

    def _num(r, carry):
        for cb in range(8):
            col = cb * 16
            sv = sd_all[r, 0, pl.ds(col, 16)]
            dv = sd_all[r, 1, pl.ds(col, 16)]
            e = plsc.load_gather(es_v, [sv]) + plsc.load_gather(ed_v, [dv])
            e = jnp.maximum(e, NEG_SLOPE * e)
            p_all[r, 0, pl.ds(col, 16)] = jnp.exp(e - mvec)
        return carry

    lax.fori_loop(0, nrows, _num, 0)

    def _scat(k, carry):
        for q in range(8):
            pltpu.async_copy(p_all.at[k * 8 + q, 0],
                             s_sh.at[sd_all.at[k * 8 + q, 1]], ssem, add=True)
        for q in range(8):
            pltpu.make_async_copy(p_all.at[k * 8 + q, 0],
                                  s_sh.at[sd_all.at[k * 8 + q, 1]], ssem).wait()
        return carry

    lax.fori_loop(0, nchunk, _scat, 0)

    @pl.when(wid == 31)
    def _():
        for q in range(4):
            pltpu.sync_copy(p_all.at[72 + q, 0],
                            s_sh.at[sd_all.at[72 + q, 1]], add=True)

    plsc.subcore_barrier()

    @pl.when(sid < 10)
    def _():
        pltpu.sync_copy(s_sh.at[pl.ds(sid * 1000, 1000)], z_v.at[pl.ds(0, 1000)])

    @pl.when((sid < 10) & (c == 0))
    def _():
        pltpu.sync_copy(z_v.at[pl.ds(0, 1000)], sa_hbm.at[pl.ds(sid * 1000, 1000)])

    @pl.when((sid < 10) & (c == 1))
    def _():
        pltpu.sync_copy(z_v.at[pl.ds(0, 1000)], sb_hbm.at[pl.ds(sid * 1000, 1000)])

    plsc.subcore_barrier()
    pltpu.core_barrier(bsem, core_axis_name="c")

    pltpu.sync_copy(sa_hbm, es_v)
    pltpu.sync_copy(sb_hbm, ed_v)

    def _alpha(r, carry):
        for cb in range(8):
            col = cb * 16
            dv = sd_all[r, 1, pl.ds(col, 16)]
            den = plsc.load_gather(es_v, [dv]) + plsc.load_gather(ed_v, [dv])
            p_all[r, 0, pl.ds(col, 16)] = (
                p_all[r, 0, pl.ds(col, 16)] / (den + EPS))
        return carry

    lax.fori_loop(0, nrows, _alpha, 0)

    pltpu.sync_copy(p_all.at[pl.ds(0, 72)], al_hbm.at[pl.ds(base, 72)])

    @pl.when(wid < 24)
    def _():
        pltpu.sync_copy(p_all.at[pl.ds(72, 8)], al_hbm.at[pl.ds(base + 72, 8)])

    @pl.when(wid == 31)
    def _():
        pltpu.sync_copy(p_all.at[pl.ds(72, 4)], al_hbm.at[pl.ds(2496, 4)])


_pass_a = pl.kernel(
    _pass_a_body,
    out_type=[
        jax.ShapeDtypeStruct((ROWS_E, 1, 128), jnp.float32),
        jax.ShapeDtypeStruct((N,), jnp.float32),
        jax.ShapeDtypeStruct((N,), jnp.float32),
    ],
    mesh=_mesh,
    compiler_params=pltpu.CompilerParams(needs_layout_passes=False),
    scratch_types=[
        pltpu.VMEM((N,), jnp.float32),
        pltpu.VMEM((N,), jnp.float32),
        pltpu.VMEM((16,), jnp.float32),
        pltpu.VMEM((80, 2, 128), jnp.int32),
        pltpu.VMEM((80, 1, 128), jnp.float32),
        pltpu.VMEM((1024,), jnp.float32),
        pltpu.SemaphoreType.DMA,
        pltpu.SemaphoreType.REGULAR,
        pltpu.VMEM_SHARED((N,), jnp.float32),
    ],
)


def _pass_b_body(sd_hbm, al_hbm, h_hbm,
                 oa_hbm, ob_hbm,
                 sd8_v, al8_v, rows0_v, rows1_v,
                 gsem0, gsem1, ssem0, ssem1, o_sh):
    c = lax.axis_index("c")
    sid = lax.axis_index("s")
    wid = sid * 2 + c
    nblk = jnp.where(wid < 24, 10, 9)
    base_blk = jnp.where(wid < 24, wid * 10, 240 + (wid - 24) * 9)

    zero16 = jnp.zeros((16,), jnp.float32)
    zi16 = jnp.zeros((16,), jnp.int32)

    def _zb(i, carry):
        rows0_v[i // 8, pl.ds((i % 8) * 16, 16)] = zero16
        return carry

    lax.fori_loop(0, 128 * 8, _zb, 0)

    @pl.when(sid < 10)
    def _():
        for q in range(7):
            pltpu.sync_copy(rows0_v, o_sh.at[pl.ds(sid * 1000 + q * 128, 128)])
        pltpu.sync_copy(rows0_v.at[pl.ds(0, 104)],
                        o_sh.at[pl.ds(sid * 1000 + 896, 104)])

    plsc.subcore_barrier()

    rows = (rows0_v, rows1_v)
    gsems = (gsem0, gsem1)
    ssems = (ssem0, ssem1)

    def _fetch_blk(j, par):
        row0 = (base_blk + j) * 8
        pltpu.sync_copy(sd_hbm.at[pl.ds(row0, 8)], sd8_v.at[par])
        pltpu.sync_copy(al_hbm.at[pl.ds(row0, 8)], al8_v.at[par])

    def _gather(par, rr, b):
        pltpu.async_copy(h_hbm.at[sd8_v.at[par, rr, 0]], rows[b], gsems[b])

    def _wait_gather(par, rr, b):
        pltpu.make_async_copy(h_hbm.at[sd8_v.at[par, rr, 0]], rows[b],
                              gsems[b]).wait()

    def _scatter(par, rr, b):
        pltpu.async_copy(rows[b], o_sh.at[sd8_v.at[par, rr, 1]], ssems[b],
                         add=True)

    def _wait_scatter(par, rr, b):
        pltpu.make_async_copy(rows[b], o_sh.at[sd8_v.at[par, rr, 1]],
                              ssems[b]).wait()

    def _scale(par, rr, b):
        rv = rows[b]
        p16 = zi16 + par
        r16 = zi16 + rr

        def _row(q, carry2):
            asp = plsc.load_gather(al8_v, [p16, r16, zi16, zi16 + q])
            for cc in range(8):
                rv[q, pl.ds(cc * 16, 16)] = rv[q, pl.ds(cc * 16, 16)] * asp
            return carry2

        lax.fori_loop(0, 128, _row, 0)

    _fetch_blk(0, 0)
    _gather(0, 0, 0)

    def _blk(j, carry):
        par = j % 2
        nxt = 1 - par

        @pl.when(j + 1 < nblk)
        def _():
            _fetch_blk(j + 1, nxt)

        for rr in range(8):
            b = rr % 2
            bo = 1 - b
            _wait_gather(par, rr, b)
            @pl.when((j > 0) | (rr > 0))
            def _():
                _wait_scatter(par, rr, bo)
            if rr < 7:
                _gather(par, rr + 1, bo)
            else:
                @pl.when(j + 1 < nblk)
                def _():
                    _gather(nxt, 0, bo)
            _scale(par, rr, b)
            _scatter(par, rr, b)
        return carry

    lax.fori_loop(0, nblk, _blk, 0)
    _wait_scatter(0, 0, 1)

    @pl.when(wid == 31)
    def _():
        pltpu.sync_copy(sd_hbm.at[pl.ds(2496, 4)], sd8_v.at[0, pl.ds(0, 4)])
        pltpu.sync_copy(al_hbm.at[pl.ds(2496, 4)], al8_v.at[0, pl.ds(0, 4)])
        for rr in range(4):
            pltpu.sync_copy(h_hbm.at[sd8_v.at[0, rr, 0]], rows0_v)
            _scale(0, rr, 0)
            pltpu.sync_copy(rows0_v, o_sh.at[sd8_v.at[0, rr, 1]], add=True)

    plsc.subcore_barrier()

    for q in range(8):
        nrow = 128 if q < 7 else 104

        @pl.when(sid < 10)
        def _():
            pltpu.sync_copy(o_sh.at[pl.ds(sid * 1000 + q * 128, nrow)],
                            rows0_v.at[pl.ds(0, nrow)])

        @pl.when((sid < 10) & (c == 0))
        def _():
            pltpu.sync_copy(rows0_v.at[pl.ds(0, nrow)],
                            oa_hbm.at[pl.ds(sid * 1000 + q * 128, nrow)])

        @pl.when((sid < 10) & (c == 1))
        def _():
            pltpu.sync_copy(rows0_v.at[pl.ds(0, nrow)],
                            ob_hbm.at[pl.ds(sid * 1000 + q * 128, nrow)])


_pass_b = pl.kernel(
    _pass_b_body,
    out_type=[
        jax.ShapeDtypeStruct((N, F), jnp.float32),
        jax.ShapeDtypeStruct((N, F), jnp.float32),
    ],
    mesh=_mesh,
    compiler_params=pltpu.CompilerParams(needs_layout_passes=False),
    scratch_types=[
        pltpu.VMEM((2, 8, 2, 128), jnp.int32),
        pltpu.VMEM((2, 8, 1, 128), jnp.float32),
        pltpu.VMEM((128, F), jnp.float32),
        pltpu.VMEM((128, F), jnp.float32),
        pltpu.SemaphoreType.DMA,
        pltpu.SemaphoreType.DMA,
        pltpu.SemaphoreType.DMA,
        pltpu.SemaphoreType.DMA,
        pltpu.VMEM_SHARED((N, F), jnp.float32),
    ],
)


def kernel(x, edge_index, batch, device, W1, a_src1, a_dst1, b1,
           W2, a_src2, a_dst2, b2, W_red, b_red, W_cls, b_cls):
    sd = jnp.transpose(edge_index.reshape(2, ROWS_E, 128), (1, 0, 2))

    h1, es1, ed1, m1 = _head1(x, W1, a_src1, a_dst1)
    a1, _sa1, _sb1 = _pass_a(sd, es1, ed1, m1)
    oa1, ob1 = _pass_b(sd, a1, h1)

    h2, es2, ed2, m2 = _head2(oa1, ob1, b1, W2, a_src2, a_dst2)
    a2, _sa2, _sb2 = _pass_a(sd, es2, ed2, m2)
    oa2, ob2 = _pass_b(sd, a2, h2)

    return _tail(oa2, ob2, b2, W_red, b_red, W_cls.T, b_cls)


# speedup vs baseline: 46.2101x; 1.1615x over previous
import jax
import jax.numpy as jnp
from jax import lax
from jax.experimental import pallas as pl
from jax.experimental.pallas import tpu as pltpu
from jax.experimental.pallas import tpu_sc as plsc

N = 10000
E = 320000
F = 128
ROWS_E = E // 128
ROWS_PW = 80
CH = 20
NEG_SLOPE = 0.2
EPS = 1e-16


def _head1_body(x_ref, w_ref, asrc_ref, adst_ref, h_ref, es_ref, ed_ref, m_ref):
    h = jnp.dot(x_ref[...], w_ref[...], preferred_element_type=jnp.float32)
    h_ref[...] = h
    es = jnp.sum(h * asrc_ref[...][None, :], axis=1)
    ed = jnp.sum(h * adst_ref[...][None, :], axis=1)
    es_ref[...] = es
    ed_ref[...] = ed
    m_ref[...] = jnp.full((16,), jnp.max(es) + jnp.max(ed), jnp.float32)


def _head2_body(oa_ref, ob_ref, b_ref, w_ref, asrc_ref, adst_ref,
                h_ref, es_ref, ed_ref, m_ref):
    hin = jnp.maximum(oa_ref[...] + ob_ref[...] + b_ref[...][None, :], 0.0)
    h = jnp.dot(hin, w_ref[...], preferred_element_type=jnp.float32)
    h_ref[...] = h
    es = jnp.sum(h * asrc_ref[...][None, :], axis=1)
    ed = jnp.sum(h * adst_ref[...][None, :], axis=1)
    es_ref[...] = es
    ed_ref[...] = ed
    m_ref[...] = jnp.full((16,), jnp.max(es) + jnp.max(ed), jnp.float32)


def _tail_body(oa_ref, ob_ref, b_ref, wred_ref, bred_ref, wclsT_ref, bcls_ref,
               y_ref):
    g = jnp.maximum(oa_ref[...] + ob_ref[...] + b_ref[...][None, :], 0.0)
    z = jnp.dot(g, wred_ref[...], preferred_element_type=jnp.float32)[:, 0]
    z = z + bred_ref[0]
    y = jnp.dot(wclsT_ref[...], z, preferred_element_type=jnp.float32)
    y_ref[...] = (y + bcls_ref[...]).reshape(1, 2)


_head1 = pl.pallas_call(
    _head1_body,
    out_shape=[
        jax.ShapeDtypeStruct((N, F), jnp.float32),
        jax.ShapeDtypeStruct((N,), jnp.float32),
        jax.ShapeDtypeStruct((N,), jnp.float32),
        jax.ShapeDtypeStruct((16,), jnp.float32),
    ],
)

_head2 = pl.pallas_call(
    _head2_body,
    out_shape=[
        jax.ShapeDtypeStruct((N, F), jnp.float32),
        jax.ShapeDtypeStruct((N,), jnp.float32),
        jax.ShapeDtypeStruct((N,), jnp.float32),
        jax.ShapeDtypeStruct((16,), jnp.float32),
    ],
)

_tail = pl.pallas_call(
    _tail_body,
    out_shape=jax.ShapeDtypeStruct((1, 2), jnp.float32),
)


_mesh = plsc.VectorSubcoreMesh(core_axis_name="c", subcore_axis_name="s")


def _wid_and_trips():
    c = lax.axis_index("c")
    sid = lax.axis_index("s")
    wid = sid * 2 + c
    trips = jnp.where(wid == 31, 5, ROWS_PW // CH)
    return c, sid, wid, trips


def _pass_a_body(sd_hbm, es_hbm, ed_hbm, m_hbm,
                 al_hbm, sa_hbm, sb_hbm,
                 es_v, ed_v, m_v, sd_all, p_all, z_v, ssem, bsem, s_sh):
    c = lax.axis_index("c")
    sid = lax.axis_index("s")
    wid = sid * 2 + c
    nrows = jnp.where(wid < 24, 80, jnp.where(wid < 31, 72, 76))
    base = jnp.where(wid < 24, wid * 80, 1920 + (wid - 24) * 72)
    nchunk = jnp.where(wid < 24, 10, 9)

    pltpu.sync_copy(es_hbm, es_v)
    pltpu.sync_copy(ed_hbm, ed_v)
    pltpu.sync_copy(m_hbm, m_v)
    pltpu.sync_copy(sd_hbm.at[pl.ds(base, 72)], sd_all.at[pl.ds(0, 72)])

    @pl.when(wid < 24)
    def _():
        pltpu.sync_copy(sd_hbm.at[pl.ds(base + 72, 8)],
                        sd_all.at[pl.ds(72, 8)])

    @pl.when(wid == 31)
    def _():
        pltpu.sync_copy(sd_hbm.at[pl.ds(2496, 4)], sd_all.at[pl.ds(72, 4)])

    zero16 = jnp.zeros((16,), jnp.float32)

    def _zb(i, carry):
        z_v[pl.ds(i * 16, 16)] = zero16
        return carry

    lax.fori_loop(0, 64, _zb, 0)

    @pl.when(sid < 10)
    def _():
        pltpu.sync_copy(z_v.at[pl.ds(0, 1000)], s_sh.at[pl.ds(sid * 1000, 1000)])

    plsc.subcore_barrier()

    mvec = m_v[...]

    def _num(r, carry):
        for cb in range(8):
            col = cb * 16
            sv = sd_all[r, 0, pl.ds(col, 16)]
            dv = sd_all[r, 1, pl.ds(col, 16)]
            e = plsc.load_gather(es_v, [sv]) + plsc.load_gather(ed_v, [dv])
            e = jnp.maximum(e, NEG_SLOPE * e)
            p_all[r, 0, pl.ds(col, 16)] = jnp.exp(e - mvec)
        return carry

    lax.fori_loop(0, nrows, _num, 0)

    def _scat(k, carry):
        for q in range(8):
            pltpu.async_copy(p_all.at[k * 8 + q, 0],
                             s_sh.at[sd_all.at[k * 8 + q, 1]], ssem, add=True)
        for q in range(8):
            pltpu.make_async_copy(p_all.at[k * 8 + q, 0],
                                  s_sh.at[sd_all.at[k * 8 + q, 1]], ssem).wait()
        return carry

    lax.fori_loop(0, nchunk, _scat, 0)

    @pl.when(wid == 31)
    def _():
        for q in range(4):
            pltpu.sync_copy(p_all.at[72 + q, 0],
                            s_sh.at[sd_all.at[72 + q, 1]], add=True)

    plsc.subcore_barrier()

    @pl.when(sid < 10)
    def _():
        pltpu.sync_copy(s_sh.at[pl.ds(sid * 1000, 1000)], z_v.at[pl.ds(0, 1000)])

    @pl.when((sid < 10) & (c == 0))
    def _():
        pltpu.sync_copy(z_v.at[pl.ds(0, 1000)], sa_hbm.at[pl.ds(sid * 1000, 1000)])

    @pl.when((sid < 10) & (c == 1))
    def _():
        pltpu.sync_copy(z_v.at[pl.ds(0, 1000)], sb_hbm.at[pl.ds(sid * 1000, 1000)])

    plsc.subcore_barrier()
    pltpu.core_barrier(bsem, core_axis_name="c")

    pltpu.sync_copy(sa_hbm, es_v)
    pltpu.sync_copy(sb_hbm, ed_v)

    def _alpha(r, carry):
        for cb in range(8):
            col = cb * 16
            dv = sd_all[r, 1, pl.ds(col, 16)]
            den = plsc.load_gather(es_v, [dv]) + plsc.load_gather(ed_v, [dv])
            p_all[r, 0, pl.ds(col, 16)] = (
                p_all[r, 0, pl.ds(col, 16)] / (den + EPS))
        return carry

    lax.fori_loop(0, nrows, _alpha, 0)

    pltpu.sync_copy(p_all.at[pl.ds(0, 72)], al_hbm.at[pl.ds(base, 72)])

    @pl.when(wid < 24)
    def _():
        pltpu.sync_copy(p_all.at[pl.ds(72, 8)], al_hbm.at[pl.ds(base + 72, 8)])

    @pl.when(wid == 31)
    def _():
        pltpu.sync_copy(p_all.at[pl.ds(72, 4)], al_hbm.at[pl.ds(2496, 4)])


_pass_a = pl.kernel(
    _pass_a_body,
    out_type=[
        jax.ShapeDtypeStruct((ROWS_E, 1, 128), jnp.float32),
        jax.ShapeDtypeStruct((N,), jnp.float32),
        jax.ShapeDtypeStruct((N,), jnp.float32),
    ],
    mesh=_mesh,
    compiler_params=pltpu.CompilerParams(needs_layout_passes=False),
    scratch_types=[
        pltpu.VMEM((N,), jnp.float32),
        pltpu.VMEM((N,), jnp.float32),
        pltpu.VMEM((16,), jnp.float32),
        pltpu.VMEM((80, 2, 128), jnp.int32),
        pltpu.VMEM((80, 1, 128), jnp.float32),
        pltpu.VMEM((1024,), jnp.float32),
        pltpu.SemaphoreType.DMA,
        pltpu.SemaphoreType.REGULAR,
        pltpu.VMEM_SHARED((N,), jnp.float32),
    ],
)


def _pass_b_body(sd_hbm, al_hbm, h_hbm,
                 oa_hbm, ob_hbm,
                 sd8_v, al8_v, rows0_v, rows1_v,
                 gsem0, gsem1, ssem0, ssem1, o_sh):
    c = lax.axis_index("c")
    sid = lax.axis_index("s")
    wid = sid * 2 + c
    nblk = jnp.where(wid < 24, 10, 9)
    base_blk = jnp.where(wid < 24, wid * 10, 240 + (wid - 24) * 9)

    zero16 = jnp.zeros((16,), jnp.float32)
    zi16 = jnp.zeros((16,), jnp.int32)

    def _zb(i, carry):
        rows0_v[i // 8, pl.ds((i % 8) * 16, 16)] = zero16
        return carry

    lax.fori_loop(0, 128 * 8, _zb, 0)

    @pl.when(sid < 10)
    def _():
        for q in range(7):
            pltpu.sync_copy(rows0_v, o_sh.at[pl.ds(sid * 1000 + q * 128, 128)])
        pltpu.sync_copy(rows0_v.at[pl.ds(0, 104)],
                        o_sh.at[pl.ds(sid * 1000 + 896, 104)])

    plsc.subcore_barrier()

    rows = (rows0_v, rows1_v)
    gsems = (gsem0, gsem1)
    ssems = (ssem0, ssem1)

    def _fetch_blk(j, par):
        row0 = (base_blk + j) * 8
        pltpu.sync_copy(sd_hbm.at[pl.ds(row0, 8)], sd8_v.at[par])
        pltpu.sync_copy(al_hbm.at[pl.ds(row0, 8)], al8_v.at[par])

    def _gather(par, rr, b):
        pltpu.async_copy(h_hbm.at[sd8_v.at[par, rr, 0]], rows[b], gsems[b])

    def _wait_gather(par, rr, b):
        pltpu.make_async_copy(h_hbm.at[sd8_v.at[par, rr, 0]], rows[b],
                              gsems[b]).wait()

    def _scatter(par, rr, b):
        pltpu.async_copy(rows[b], o_sh.at[sd8_v.at[par, rr, 1]], ssems[b],
                         add=True)

    def _wait_scatter(par, rr, b):
        pltpu.make_async_copy(rows[b], o_sh.at[sd8_v.at[par, rr, 1]],
                              ssems[b]).wait()

    def _scale(par, rr, b):
        rv = rows[b]
        p16 = zi16 + par
        r16 = zi16 + rr

        @plsc.parallel_loop(0, 128, 1, unroll=4)
        def _row(q):
            asp = plsc.load_gather(al8_v, [p16, r16, zi16, zi16 + q])
            for cc in range(8):
                rv[q, pl.ds(cc * 16, 16)] = rv[q, pl.ds(cc * 16, 16)] * asp

    _fetch_blk(0, 0)
    _gather(0, 0, 0)

    def _blk(j, carry):
        par = j % 2
        nxt = 1 - par

        @pl.when(j + 1 < nblk)
        def _():
            _fetch_blk(j + 1, nxt)

        for rr in range(8):
            b = rr % 2
            bo = 1 - b
            _wait_gather(par, rr, b)
            @pl.when((j > 0) | (rr > 0))
            def _():
                _wait_scatter(par, rr, bo)
            if rr < 7:
                _gather(par, rr + 1, bo)
            else:
                @pl.when(j + 1 < nblk)
                def _():
                    _gather(nxt, 0, bo)
            _scale(par, rr, b)
            _scatter(par, rr, b)
        return carry

    lax.fori_loop(0, nblk, _blk, 0)
    _wait_scatter(0, 0, 1)

    @pl.when(wid == 31)
    def _():
        pltpu.sync_copy(sd_hbm.at[pl.ds(2496, 4)], sd8_v.at[0, pl.ds(0, 4)])
        pltpu.sync_copy(al_hbm.at[pl.ds(2496, 4)], al8_v.at[0, pl.ds(0, 4)])
        for rr in range(4):
            pltpu.sync_copy(h_hbm.at[sd8_v.at[0, rr, 0]], rows0_v)
            _scale(0, rr, 0)
            pltpu.sync_copy(rows0_v, o_sh.at[sd8_v.at[0, rr, 1]], add=True)

    plsc.subcore_barrier()

    for q in range(8):
        nrow = 128 if q < 7 else 104

        @pl.when(sid < 10)
        def _():
            pltpu.sync_copy(o_sh.at[pl.ds(sid * 1000 + q * 128, nrow)],
                            rows0_v.at[pl.ds(0, nrow)])

        @pl.when((sid < 10) & (c == 0))
        def _():
            pltpu.sync_copy(rows0_v.at[pl.ds(0, nrow)],
                            oa_hbm.at[pl.ds(sid * 1000 + q * 128, nrow)])

        @pl.when((sid < 10) & (c == 1))
        def _():
            pltpu.sync_copy(rows0_v.at[pl.ds(0, nrow)],
                            ob_hbm.at[pl.ds(sid * 1000 + q * 128, nrow)])


_pass_b = pl.kernel(
    _pass_b_body,
    out_type=[
        jax.ShapeDtypeStruct((N, F), jnp.float32),
        jax.ShapeDtypeStruct((N, F), jnp.float32),
    ],
    mesh=_mesh,
    compiler_params=pltpu.CompilerParams(needs_layout_passes=False),
    scratch_types=[
        pltpu.VMEM((2, 8, 2, 128), jnp.int32),
        pltpu.VMEM((2, 8, 1, 128), jnp.float32),
        pltpu.VMEM((128, F), jnp.float32),
        pltpu.VMEM((128, F), jnp.float32),
        pltpu.SemaphoreType.DMA,
        pltpu.SemaphoreType.DMA,
        pltpu.SemaphoreType.DMA,
        pltpu.SemaphoreType.DMA,
        pltpu.VMEM_SHARED((N, F), jnp.float32),
    ],
)


def kernel(x, edge_index, batch, device, W1, a_src1, a_dst1, b1,
           W2, a_src2, a_dst2, b2, W_red, b_red, W_cls, b_cls):
    sd = jnp.transpose(edge_index.reshape(2, ROWS_E, 128), (1, 0, 2))

    h1, es1, ed1, m1 = _head1(x, W1, a_src1, a_dst1)
    a1, _sa1, _sb1 = _pass_a(sd, es1, ed1, m1)
    oa1, ob1 = _pass_b(sd, a1, h1)

    h2, es2, ed2, m2 = _head2(oa1, ob1, b1, W2, a_src2, a_dst2)
    a2, _sa2, _sb2 = _pass_a(sd, es2, ed2, m2)
    oa2, ob2 = _pass_b(sd, a2, h2)

    return _tail(oa2, ob2, b2, W_red, b_red, W_cls.T, b_cls)


# speedup vs baseline: 51.6244x; 1.1172x over previous
import jax
import jax.numpy as jnp
from jax import lax
from jax.experimental import pallas as pl
from jax.experimental.pallas import tpu as pltpu
from jax.experimental.pallas import tpu_sc as plsc

N = 10000
E = 320000
F = 128
ROWS_E = E // 128
ROWS_PW = 80
CH = 20
NEG_SLOPE = 0.2
EPS = 1e-16


def _head1_body(x_ref, w_ref, asrc_ref, adst_ref, h_ref, es_ref, ed_ref, m_ref):
    h = jnp.dot(x_ref[...], w_ref[...], preferred_element_type=jnp.float32)
    h_ref[...] = h
    es = jnp.sum(h * asrc_ref[...][None, :], axis=1)
    ed = jnp.sum(h * adst_ref[...][None, :], axis=1)
    es_ref[...] = es
    ed_ref[...] = ed
    m_ref[...] = jnp.full((16,), jnp.max(es) + jnp.max(ed), jnp.float32)


def _head2_body(oa_ref, ob_ref, b_ref, w_ref, asrc_ref, adst_ref,
                h_ref, es_ref, ed_ref, m_ref):
    hin = jnp.maximum(oa_ref[...] + ob_ref[...] + b_ref[...][None, :], 0.0)
    h = jnp.dot(hin, w_ref[...], preferred_element_type=jnp.float32)
    h_ref[...] = h
    es = jnp.sum(h * asrc_ref[...][None, :], axis=1)
    ed = jnp.sum(h * adst_ref[...][None, :], axis=1)
    es_ref[...] = es
    ed_ref[...] = ed
    m_ref[...] = jnp.full((16,), jnp.max(es) + jnp.max(ed), jnp.float32)


def _tail_body(oa_ref, ob_ref, b_ref, wred_ref, bred_ref, wclsT_ref, bcls_ref,
               y_ref):
    g = jnp.maximum(oa_ref[...] + ob_ref[...] + b_ref[...][None, :], 0.0)
    z = jnp.dot(g, wred_ref[...], preferred_element_type=jnp.float32)[:, 0]
    z = z + bred_ref[0]
    y = jnp.dot(wclsT_ref[...], z, preferred_element_type=jnp.float32)
    y_ref[...] = (y + bcls_ref[...]).reshape(1, 2)


_head1 = pl.pallas_call(
    _head1_body,
    out_shape=[
        jax.ShapeDtypeStruct((N, F), jnp.float32),
        jax.ShapeDtypeStruct((N,), jnp.float32),
        jax.ShapeDtypeStruct((N,), jnp.float32),
        jax.ShapeDtypeStruct((16,), jnp.float32),
    ],
)

_head2 = pl.pallas_call(
    _head2_body,
    out_shape=[
        jax.ShapeDtypeStruct((N, F), jnp.float32),
        jax.ShapeDtypeStruct((N,), jnp.float32),
        jax.ShapeDtypeStruct((N,), jnp.float32),
        jax.ShapeDtypeStruct((16,), jnp.float32),
    ],
)

_tail = pl.pallas_call(
    _tail_body,
    out_shape=jax.ShapeDtypeStruct((1, 2), jnp.float32),
)


_mesh = plsc.VectorSubcoreMesh(core_axis_name="c", subcore_axis_name="s")


def _wid_and_trips():
    c = lax.axis_index("c")
    sid = lax.axis_index("s")
    wid = sid * 2 + c
    trips = jnp.where(wid == 31, 5, ROWS_PW // CH)
    return c, sid, wid, trips


def _pass_a_body(sd_hbm, es_hbm, ed_hbm, m_hbm,
                 al_hbm, sa_hbm, sb_hbm,
                 es_v, ed_v, m_v, sd_all, p_all, z_v, ssem, bsem, s_sh):
    c = lax.axis_index("c")
    sid = lax.axis_index("s")
    wid = sid * 2 + c
    nrows = jnp.where(wid < 24, 80, jnp.where(wid < 31, 72, 76))
    base = jnp.where(wid < 24, wid * 80, 1920 + (wid - 24) * 72)
    nchunk = jnp.where(wid < 24, 10, 9)

    pltpu.sync_copy(es_hbm, es_v)
    pltpu.sync_copy(ed_hbm, ed_v)
    pltpu.sync_copy(m_hbm, m_v)
    pltpu.sync_copy(sd_hbm.at[pl.ds(base, 72)], sd_all.at[pl.ds(0, 72)])

    @pl.when(wid < 24)
    def _():
        pltpu.sync_copy(sd_hbm.at[pl.ds(base + 72, 8)],
                        sd_all.at[pl.ds(72, 8)])

    @pl.when(wid == 31)
    def _():
        pltpu.sync_copy(sd_hbm.at[pl.ds(2496, 4)], sd_all.at[pl.ds(72, 4)])

    zero16 = jnp.zeros((16,), jnp.float32)

    def _zb(i, carry):
        z_v[pl.ds(i * 16, 16)] = zero16
        return carry

    lax.fori_loop(0, 64, _zb, 0)

    @pl.when(sid < 10)
    def _():
        pltpu.sync_copy(z_v.at[pl.ds(0, 1000)], s_sh.at[pl.ds(sid * 1000, 1000)])

    plsc.subcore_barrier()

    mvec = m_v[...]

    @plsc.parallel_loop(0, nrows, 1, unroll=2)
    def _num(r):
        for cb in range(8):
            col = cb * 16
            sv = sd_all[r, 0, pl.ds(col, 16)]
            dv = sd_all[r, 1, pl.ds(col, 16)]
            e = plsc.load_gather(es_v, [sv]) + plsc.load_gather(ed_v, [dv])
            e = jnp.maximum(e, NEG_SLOPE * e)
            p_all[r, 0, pl.ds(col, 16)] = jnp.exp(e - mvec)

    def _scat(k, carry):
        for q in range(8):
            pltpu.async_copy(p_all.at[k * 8 + q, 0],
                             s_sh.at[sd_all.at[k * 8 + q, 1]], ssem, add=True)
        for q in range(8):
            pltpu.make_async_copy(p_all.at[k * 8 + q, 0],
                                  s_sh.at[sd_all.at[k * 8 + q, 1]], ssem).wait()
        return carry

    lax.fori_loop(0, nchunk, _scat, 0)

    @pl.when(wid == 31)
    def _():
        for q in range(4):
            pltpu.sync_copy(p_all.at[72 + q, 0],
                            s_sh.at[sd_all.at[72 + q, 1]], add=True)

    plsc.subcore_barrier()

    @pl.when(sid < 10)
    def _():
        pltpu.sync_copy(s_sh.at[pl.ds(sid * 1000, 1000)], z_v.at[pl.ds(0, 1000)])

    @pl.when((sid < 10) & (c == 0))
    def _():
        pltpu.sync_copy(z_v.at[pl.ds(0, 1000)], sa_hbm.at[pl.ds(sid * 1000, 1000)])

    @pl.when((sid < 10) & (c == 1))
    def _():
        pltpu.sync_copy(z_v.at[pl.ds(0, 1000)], sb_hbm.at[pl.ds(sid * 1000, 1000)])

    plsc.subcore_barrier()
    pltpu.core_barrier(bsem, core_axis_name="c")

    pltpu.sync_copy(sa_hbm, es_v)
    pltpu.sync_copy(sb_hbm, ed_v)

    @plsc.parallel_loop(0, nrows, 1, unroll=2)
    def _alpha(r):
        for cb in range(8):
            col = cb * 16
            dv = sd_all[r, 1, pl.ds(col, 16)]
            den = plsc.load_gather(es_v, [dv]) + plsc.load_gather(ed_v, [dv])
            p_all[r, 0, pl.ds(col, 16)] = (
                p_all[r, 0, pl.ds(col, 16)] / (den + EPS))

    pltpu.sync_copy(p_all.at[pl.ds(0, 72)], al_hbm.at[pl.ds(base, 72)])

    @pl.when(wid < 24)
    def _():
        pltpu.sync_copy(p_all.at[pl.ds(72, 8)], al_hbm.at[pl.ds(base + 72, 8)])

    @pl.when(wid == 31)
    def _():
        pltpu.sync_copy(p_all.at[pl.ds(72, 4)], al_hbm.at[pl.ds(2496, 4)])


_pass_a = pl.kernel(
    _pass_a_body,
    out_type=[
        jax.ShapeDtypeStruct((ROWS_E, 1, 128), jnp.float32),
        jax.ShapeDtypeStruct((N,), jnp.float32),
        jax.ShapeDtypeStruct((N,), jnp.float32),
    ],
    mesh=_mesh,
    compiler_params=pltpu.CompilerParams(needs_layout_passes=False),
    scratch_types=[
        pltpu.VMEM((N,), jnp.float32),
        pltpu.VMEM((N,), jnp.float32),
        pltpu.VMEM((16,), jnp.float32),
        pltpu.VMEM((80, 2, 128), jnp.int32),
        pltpu.VMEM((80, 1, 128), jnp.float32),
        pltpu.VMEM((1024,), jnp.float32),
        pltpu.SemaphoreType.DMA,
        pltpu.SemaphoreType.REGULAR,
        pltpu.VMEM_SHARED((N,), jnp.float32),
    ],
)


def _pass_b_body(sd_hbm, al_hbm, h_hbm,
                 oa_hbm, ob_hbm,
                 sd8_v, al8_v, rows0_v, rows1_v,
                 gsem0, gsem1, ssem0, ssem1, o_sh):
    c = lax.axis_index("c")
    sid = lax.axis_index("s")
    wid = sid * 2 + c
    nblk = jnp.where(wid < 24, 10, 9)
    base_blk = jnp.where(wid < 24, wid * 10, 240 + (wid - 24) * 9)

    zero16 = jnp.zeros((16,), jnp.float32)
    zi16 = jnp.zeros((16,), jnp.int32)

    @plsc.parallel_loop(0, 128 * 8, 1, unroll=4)
    def _zb(i):
        rows0_v[i // 8, pl.ds((i % 8) * 16, 16)] = zero16

    @pl.when(sid < 10)
    def _():
        for q in range(7):
            pltpu.sync_copy(rows0_v, o_sh.at[pl.ds(sid * 1000 + q * 128, 128)])
        pltpu.sync_copy(rows0_v.at[pl.ds(0, 104)],
                        o_sh.at[pl.ds(sid * 1000 + 896, 104)])

    plsc.subcore_barrier()

    rows = (rows0_v, rows1_v)
    gsems = (gsem0, gsem1)
    ssems = (ssem0, ssem1)

    def _fetch_blk(j, par):
        row0 = (base_blk + j) * 8
        pltpu.sync_copy(sd_hbm.at[pl.ds(row0, 8)], sd8_v.at[par])
        pltpu.sync_copy(al_hbm.at[pl.ds(row0, 8)], al8_v.at[par])

    def _gather(par, rr, b):
        pltpu.async_copy(h_hbm.at[sd8_v.at[par, rr, 0]], rows[b], gsems[b])

    def _wait_gather(par, rr, b):
        pltpu.make_async_copy(h_hbm.at[sd8_v.at[par, rr, 0]], rows[b],
                              gsems[b]).wait()

    def _scatter(par, rr, b):
        pltpu.async_copy(rows[b], o_sh.at[sd8_v.at[par, rr, 1]], ssems[b],
                         add=True)

    def _wait_scatter(par, rr, b):
        pltpu.make_async_copy(rows[b], o_sh.at[sd8_v.at[par, rr, 1]],
                              ssems[b]).wait()

    def _scale(par, rr, b):
        rv = rows[b]
        p16 = zi16 + par
        r16 = zi16 + rr

        @plsc.parallel_loop(0, 128, 1, unroll=4)
        def _row(q):
            asp = plsc.load_gather(al8_v, [p16, r16, zi16, zi16 + q])
            for cc in range(8):
                rv[q, pl.ds(cc * 16, 16)] = rv[q, pl.ds(cc * 16, 16)] * asp

    _fetch_blk(0, 0)
    _gather(0, 0, 0)

    def _blk(j, carry):
        par = j % 2
        nxt = 1 - par

        @pl.when(j + 1 < nblk)
        def _():
            _fetch_blk(j + 1, nxt)

        for rr in range(8):
            b = rr % 2
            bo = 1 - b
            _wait_gather(par, rr, b)
            @pl.when((j > 0) | (rr > 0))
            def _():
                _wait_scatter(par, rr, bo)
            if rr < 7:
                _gather(par, rr + 1, bo)
            else:
                @pl.when(j + 1 < nblk)
                def _():
                    _gather(nxt, 0, bo)
            _scale(par, rr, b)
            _scatter(par, rr, b)
        return carry

    lax.fori_loop(0, nblk, _blk, 0)
    _wait_scatter(0, 0, 1)

    @pl.when(wid == 31)
    def _():
        pltpu.sync_copy(sd_hbm.at[pl.ds(2496, 4)], sd8_v.at[0, pl.ds(0, 4)])
        pltpu.sync_copy(al_hbm.at[pl.ds(2496, 4)], al8_v.at[0, pl.ds(0, 4)])
        for rr in range(4):
            pltpu.sync_copy(h_hbm.at[sd8_v.at[0, rr, 0]], rows0_v)
            _scale(0, rr, 0)
            pltpu.sync_copy(rows0_v, o_sh.at[sd8_v.at[0, rr, 1]], add=True)

    plsc.subcore_barrier()

    for q in range(8):
        nrow = 128 if q < 7 else 104

        @pl.when(sid < 10)
        def _():
            pltpu.sync_copy(o_sh.at[pl.ds(sid * 1000 + q * 128, nrow)],
                            rows0_v.at[pl.ds(0, nrow)])

        @pl.when((sid < 10) & (c == 0))
        def _():
            pltpu.sync_copy(rows0_v.at[pl.ds(0, nrow)],
                            oa_hbm.at[pl.ds(sid * 1000 + q * 128, nrow)])

        @pl.when((sid < 10) & (c == 1))
        def _():
            pltpu.sync_copy(rows0_v.at[pl.ds(0, nrow)],
                            ob_hbm.at[pl.ds(sid * 1000 + q * 128, nrow)])


_pass_b = pl.kernel(
    _pass_b_body,
    out_type=[
        jax.ShapeDtypeStruct((N, F), jnp.float32),
        jax.ShapeDtypeStruct((N, F), jnp.float32),
    ],
    mesh=_mesh,
    compiler_params=pltpu.CompilerParams(needs_layout_passes=False),
    scratch_types=[
        pltpu.VMEM((2, 8, 2, 128), jnp.int32),
        pltpu.VMEM((2, 8, 1, 128), jnp.float32),
        pltpu.VMEM((128, F), jnp.float32),
        pltpu.VMEM((128, F), jnp.float32),
        pltpu.SemaphoreType.DMA,
        pltpu.SemaphoreType.DMA,
        pltpu.SemaphoreType.DMA,
        pltpu.SemaphoreType.DMA,
        pltpu.VMEM_SHARED((N, F), jnp.float32),
    ],
)


def kernel(x, edge_index, batch, device, W1, a_src1, a_dst1, b1,
           W2, a_src2, a_dst2, b2, W_red, b_red, W_cls, b_cls):
    sd = jnp.transpose(edge_index.reshape(2, ROWS_E, 128), (1, 0, 2))

    h1, es1, ed1, m1 = _head1(x, W1, a_src1, a_dst1)
    a1, _sa1, _sb1 = _pass_a(sd, es1, ed1, m1)
    oa1, ob1 = _pass_b(sd, a1, h1)

    h2, es2, ed2, m2 = _head2(oa1, ob1, b1, W2, a_src2, a_dst2)
    a2, _sa2, _sb2 = _pass_a(sd, es2, ed2, m2)
    oa2, ob2 = _pass_b(sd, a2, h2)

    return _tail(oa2, ob2, b2, W_red, b_red, W_cls.T, b_cls)
